# baseline (speedup 1.0000x reference)
import jax
import jax.numpy as jnp
from jax.experimental import pallas as pl
from jax.experimental.pallas import tpu as pltpu

_TOPK = 8
_N_GROUPS = 8
_TOPK_GROUPS = 4
_ROUTE_SCALE = 2.5
_N_EXPERTS = 64
_DIM = 768
_TOKENS = 32768

_BT = 4096
_NEG = float("-inf")


def _moe_gate_block(x_ref, w_ref, wout_ref, iout_ref):
    x = x_ref[...]
    w = w_ref[...]
    st = jax.lax.dot_general(
        w, x, (((1,), (1,)), ((), ())), preferred_element_type=jnp.float32
    )
    st = jax.nn.sigmoid(st)

    row = jax.lax.broadcasted_iota(jnp.int32, (_N_EXPERTS, _BT), 0)

    m1 = st
    m2 = None
    for k in (1, 2, 4):
        bit = (row & k) == 0
        pm1 = jnp.where(bit, pltpu.roll(m1, _N_EXPERTS - k, 0), pltpu.roll(m1, k, 0))
        if m2 is None:
            m2 = jnp.minimum(m1, pm1)
        else:
            pm2 = jnp.where(bit, pltpu.roll(m2, _N_EXPERTS - k, 0), pltpu.roll(m2, k, 0))
            m2 = jnp.maximum(jnp.minimum(m1, pm1), jnp.maximum(m2, pm2))
        m1 = jnp.maximum(m1, pm1)
    gs = m1 + m2

    g = row >> 3
    rank = jnp.zeros((_N_EXPERTS, _BT), dtype=jnp.int32)
    for j in range(1, _N_GROUPS):
        other = pltpu.roll(gs, _N_EXPERTS - 8 * j, 0)
        og_lt = ((g + j) & 7) < g
        beats = (other > gs) | ((other == gs) & og_lt)
        rank = rank + jnp.where(beats, 1, 0)
    sel = rank < _TOPK_GROUPS

    masked = jnp.where(sel, st, _NEG)
    picked_v, picked_i = [], []
    for _ in range(_TOPK):
        m = jnp.max(masked, axis=0, keepdims=True)
        am = jnp.min(
            jnp.where(masked == m, row, _N_EXPERTS), axis=0, keepdims=True
        )
        picked_v.append(m)
        picked_i.append(am)
        if len(picked_v) < _TOPK:
            masked = jnp.where(row == am, _NEG, masked)

    vals = jnp.concatenate(picked_v, axis=0)
    idx = jnp.concatenate(picked_i, axis=0)
    wts = vals * (_ROUTE_SCALE / jnp.sum(vals, axis=0, keepdims=True))

    wout_ref[...] = wts
    iout_ref[...] = idx


@jax.jit
def kernel(x, weight):
    grid = (_TOKENS // _BT,)
    wout, iout = pl.pallas_call(
        _moe_gate_block,
        grid=grid,
        in_specs=[
            pl.BlockSpec((_BT, _DIM), lambda i: (i, 0)),
            pl.BlockSpec((_N_EXPERTS, _DIM), lambda i: (0, 0)),
        ],
        out_specs=[
            pl.BlockSpec((_TOPK, _BT), lambda i: (0, i)),
            pl.BlockSpec((_TOPK, _BT), lambda i: (0, i)),
        ],
        out_shape=[
            jax.ShapeDtypeStruct((_TOPK, _TOKENS), jnp.float32),
            jax.ShapeDtypeStruct((_TOPK, _TOKENS), jnp.int32),
        ],
        compiler_params=pltpu.CompilerParams(
            dimension_semantics=("arbitrary",),
        ),
    )(x, weight)
    return wout.T, iout.T
